# baseline (speedup 1.0000x reference)
import jax
import jax.numpy as jnp
from jax.experimental import pallas as pl

_FEATURE_MAP_SIZE = 256
_OUTPUT_SIZE = 7
_FILL_VALUE = 3.0
_BLOCK_ROWS = 256


def _fill_kernel(o_ref):
    o_ref[...] = jnp.full(o_ref.shape, _FILL_VALUE, dtype=jnp.float32)


def kernel(feature_maps, rois):
    n_img = rois.shape[0]
    n_rois = rois.shape[1]
    rows = n_img * n_rois
    cols = _FEATURE_MAP_SIZE * _OUTPUT_SIZE * _OUTPUT_SIZE
    block_rows = min(_BLOCK_ROWS, rows)
    grid = rows // block_rows
    out2d = pl.pallas_call(
        _fill_kernel,
        grid=(grid,),
        out_specs=pl.BlockSpec((block_rows, cols), lambda i: (i, 0)),
        out_shape=jax.ShapeDtypeStruct((rows, cols), jnp.float32),
    )()
    return out2d.reshape(n_img, n_rois, _FEATURE_MAP_SIZE, _OUTPUT_SIZE,
                         _OUTPUT_SIZE)
